# Initial kernel scaffold; baseline (speedup 1.0000x reference)
import jax
import jax.numpy as jnp
from jax.experimental import pallas as pl


def kernel(atom_features, residue_mapping, W1, b1, W2, b2):
    raise NotImplementedError("write your pallas kernel here")


# speedup vs baseline: 3.8821x; 3.8821x over previous
import jax
import jax.numpy as jnp
from jax import lax
from jax.experimental import pallas as pl
from jax.experimental.pallas import tpu as pltpu
from jax.experimental.pallas import tpu_sc as plsc

ATOM_DIM = 128
HID = 256
OUT_DIM = 20
N_ATOMS = 320000
N_RES = 10000
BLK_ATOMS = 128
N_BLOCKS = N_ATOMS // BLK_ATOMS
NC = 2
NS = 16
NW = NC * NS
R_PAD = 10240
RES_PER_SUB = R_PAD // NS


def _sc_sums_body(atoms, map1d, zrows, sums_out, data_v, idx_v, acc):
    cid = lax.axis_index("c")
    sid = lax.axis_index("s")
    wid = cid * NS + sid

    base = sid * RES_PER_SUB
    pltpu.sync_copy(zrows, data_v)
    for k in range(RES_PER_SUB // BLK_ATOMS):
        pltpu.sync_copy(data_v, acc.at[pl.ds(base + k * BLK_ATOMS, BLK_ATOMS)])
    plsc.subcore_barrier()

    def scatter_block(b):
        pltpu.sync_copy(atoms.at[pl.ds(b * BLK_ATOMS, BLK_ATOMS)], data_v)
        pltpu.sync_copy(map1d.at[pl.ds(b * BLK_ATOMS, BLK_ATOMS)], idx_v)
        pltpu.sync_copy(data_v, acc.at[idx_v], add=True)

    def step(i, carry):
        scatter_block(wid + i * NW)
        return carry

    lax.fori_loop(0, N_BLOCKS // NW, step, 0)

    @pl.when(wid < N_BLOCKS - (N_BLOCKS // NW) * NW)
    def _():
        scatter_block((N_BLOCKS // NW) * NW + wid)

    plsc.subcore_barrier()

    out_base = cid * R_PAD + base
    for k in range(RES_PER_SUB // BLK_ATOMS):
        pltpu.sync_copy(acc.at[pl.ds(base + k * BLK_ATOMS, BLK_ATOMS)], data_v)
        pltpu.sync_copy(data_v, sums_out.at[pl.ds(out_base + k * BLK_ATOMS, BLK_ATOMS)])


def _sc_counts_body(map1d, zrows, ones_in, cnts_out, ones_v, idx_v, ccnt):
    cid = lax.axis_index("c")
    sid = lax.axis_index("s")
    wid = cid * NS + sid

    base = sid * RES_PER_SUB
    pltpu.sync_copy(zrows, ones_v)
    for k in range(RES_PER_SUB // BLK_ATOMS):
        pltpu.sync_copy(ones_v, ccnt.at[pl.ds(base + k * BLK_ATOMS, BLK_ATOMS)])
    pltpu.sync_copy(ones_in, ones_v)
    plsc.subcore_barrier()

    def scatter_block(b):
        pltpu.sync_copy(map1d.at[pl.ds(b * BLK_ATOMS, BLK_ATOMS)], idx_v)
        pltpu.sync_copy(ones_v, ccnt.at[idx_v], add=True)

    def step(i, carry):
        scatter_block(wid + i * NW)
        return carry

    lax.fori_loop(0, N_BLOCKS // NW, step, 0)

    @pl.when(wid < N_BLOCKS - (N_BLOCKS // NW) * NW)
    def _():
        scatter_block((N_BLOCKS // NW) * NW + wid)

    plsc.subcore_barrier()

    out_base = cid * R_PAD + base
    for k in range(RES_PER_SUB // BLK_ATOMS):
        pltpu.sync_copy(ccnt.at[pl.ds(base + k * BLK_ATOMS, BLK_ATOMS)], ones_v)
        pltpu.sync_copy(ones_v, cnts_out.at[pl.ds(out_base + k * BLK_ATOMS, BLK_ATOMS)])


@jax.jit
def _sc_segment_sums(atoms, map1d, zrows, ones_in):
    mesh = plsc.VectorSubcoreMesh(core_axis_name="c", subcore_axis_name="s",
                                  num_cores=NC, num_subcores=NS)
    sums = pl.kernel(
        _sc_sums_body,
        out_type=[
            jax.ShapeDtypeStruct((NC * R_PAD, ATOM_DIM), jnp.float32),
        ],
        mesh=mesh,
        scratch_types=[
            pltpu.VMEM((BLK_ATOMS, ATOM_DIM), jnp.float32),
            pltpu.VMEM((BLK_ATOMS,), jnp.int32),
            pltpu.VMEM_SHARED((R_PAD, ATOM_DIM), jnp.float32),
        ],
    )(atoms, map1d, zrows)[0]
    cnts = pl.kernel(
        _sc_counts_body,
        out_type=[
            jax.ShapeDtypeStruct((NC * R_PAD, ATOM_DIM), jnp.float32),
        ],
        mesh=mesh,
        scratch_types=[
            pltpu.VMEM((BLK_ATOMS, ATOM_DIM), jnp.float32),
            pltpu.VMEM((BLK_ATOMS,), jnp.int32),
            pltpu.VMEM_SHARED((R_PAD, ATOM_DIM), jnp.float32),
        ],
    )(map1d, zrows, ones_in)[0]
    return sums, cnts


def _mlp_body(sums_ref, cnts_ref, w1t_ref, b1_ref, w2t_ref, b2_ref, out_ref):
    s = sums_ref[0] + sums_ref[1]
    c = cnts_ref[0] + cnts_ref[1]
    r = s / c
    h = jnp.dot(r, w1t_ref[...], preferred_element_type=jnp.float32) + b1_ref[...]
    h = jnp.where(h >= 0.0, h, 0.01 * h)
    out_ref[...] = (jnp.dot(h, w2t_ref[...], preferred_element_type=jnp.float32)
                    + b2_ref[...])


MLP_BLK = 1000


@jax.jit
def _tc_mlp(sums, cnts, w1t, b1, w2t, b2):
    grid = (N_RES // MLP_BLK,)
    return pl.pallas_call(
        _mlp_body,
        grid=grid,
        in_specs=[
            pl.BlockSpec((NC, MLP_BLK, ATOM_DIM), lambda i: (0, i, 0)),
            pl.BlockSpec((NC, MLP_BLK, 1), lambda i: (0, i, 0)),
            pl.BlockSpec((ATOM_DIM, HID), lambda i: (0, 0)),
            pl.BlockSpec((1, HID), lambda i: (0, 0)),
            pl.BlockSpec((HID, OUT_DIM), lambda i: (0, 0)),
            pl.BlockSpec((1, OUT_DIM), lambda i: (0, 0)),
        ],
        out_specs=pl.BlockSpec((MLP_BLK, OUT_DIM), lambda i: (i, 0)),
        out_shape=jax.ShapeDtypeStruct((N_RES, OUT_DIM), jnp.float32),
    )(sums, cnts, w1t, b1, w2t, b2)


def kernel(atom_features, residue_mapping, W1, b1, W2, b2):
    map1d = residue_mapping.astype(jnp.int32)
    zrows = jnp.zeros((BLK_ATOMS, ATOM_DIM), jnp.float32)
    ones_in = jnp.ones((BLK_ATOMS, ATOM_DIM), jnp.float32)
    sums, cnts = _sc_segment_sums(atom_features, map1d, zrows, ones_in)
    sums = sums.reshape(NC, R_PAD, ATOM_DIM)[:, :N_RES, :]
    cnts = cnts.reshape(NC, R_PAD, ATOM_DIM)[:, :N_RES, :1]
    out = _tc_mlp(sums, cnts,
                  W1.T, b1.reshape(1, HID), W2.T, b2.reshape(1, OUT_DIM))
    return out


# speedup vs baseline: 5.7642x; 1.4848x over previous
import jax
import jax.numpy as jnp
from jax import lax
from jax.experimental import pallas as pl
from jax.experimental.pallas import tpu as pltpu
from jax.experimental.pallas import tpu_sc as plsc

ATOM_DIM = 128
HID = 256
OUT_DIM = 20
N_ATOMS = 320000
N_RES = 10000
BLK_ATOMS = 128
N_BLOCKS = N_ATOMS // BLK_ATOMS
NC = 2
NS = 16
NW = NC * NS
R_PAD = 10240
RES_PER_SUB = R_PAD // NS


def _sc_sums_body(atoms, map1d, zrows, sums_out,
                  d0, d1, i0, i1, sd0, sd1, si0, si1, acc):
    cid = lax.axis_index("c")
    sid = lax.axis_index("s")
    wid = cid * NS + sid
    bufs = ((d0, i0, sd0, si0), (d1, i1, sd1, si1))

    base = sid * RES_PER_SUB
    pltpu.sync_copy(zrows, d0)
    for k in range(RES_PER_SUB // BLK_ATOMS):
        pltpu.sync_copy(d0, acc.at[pl.ds(base + k * BLK_ATOMS, BLK_ATOMS)])
    plsc.subcore_barrier()

    def start(b, buf):
        dv, iv, sd, si = buf
        pltpu.async_copy(atoms.at[pl.ds(b * BLK_ATOMS, BLK_ATOMS)], dv, sd)
        pltpu.async_copy(map1d.at[pl.ds(b * BLK_ATOMS, BLK_ATOMS)], iv, si)

    def finish(b, buf):
        dv, iv, sd, si = buf
        pltpu.make_async_copy(atoms.at[pl.ds(b * BLK_ATOMS, BLK_ATOMS)], dv, sd).wait()
        pltpu.make_async_copy(map1d.at[pl.ds(b * BLK_ATOMS, BLK_ATOMS)], iv, si).wait()
        pltpu.sync_copy(dv, acc.at[iv], add=True)

    start(wid, bufs[0])
    start(wid + NW, bufs[1])

    def step(g, carry):
        for k in range(2):
            b = wid + (2 * g + k) * NW
            finish(b, bufs[k])

            @pl.when(b + 2 * NW < N_BLOCKS)
            def _():
                start(b + 2 * NW, bufs[k])

        return carry

    lax.fori_loop(0, N_BLOCKS // NW // 2, step, 0)

    @pl.when(wid < N_BLOCKS - (N_BLOCKS // NW) * NW)
    def _():
        finish(wid + (N_BLOCKS // NW) * NW, bufs[0])

    plsc.subcore_barrier()

    out_base = cid * R_PAD + base
    for k in range(RES_PER_SUB // BLK_ATOMS):
        pltpu.sync_copy(acc.at[pl.ds(base + k * BLK_ATOMS, BLK_ATOMS)], d0)
        pltpu.sync_copy(d0, sums_out.at[pl.ds(out_base + k * BLK_ATOMS, BLK_ATOMS)])


def _sc_counts_body(map1d, zrows, ones_in, cnts_out, ones_v, i0, i1, si0, si1, ccnt):
    cid = lax.axis_index("c")
    sid = lax.axis_index("s")
    wid = cid * NS + sid

    base = sid * RES_PER_SUB
    pltpu.sync_copy(zrows, ones_v)
    for k in range(RES_PER_SUB // BLK_ATOMS):
        pltpu.sync_copy(ones_v, ccnt.at[pl.ds(base + k * BLK_ATOMS, BLK_ATOMS)])
    pltpu.sync_copy(ones_in, ones_v)
    plsc.subcore_barrier()

    def start(b, iv, si):
        pltpu.async_copy(map1d.at[pl.ds(b * BLK_ATOMS, BLK_ATOMS)], iv, si)

    def finish(b, iv, si):
        pltpu.make_async_copy(map1d.at[pl.ds(b * BLK_ATOMS, BLK_ATOMS)], iv, si).wait()
        pltpu.sync_copy(ones_v, ccnt.at[iv], add=True)

    ibufs = ((i0, si0), (i1, si1))
    start(wid, *ibufs[0])
    start(wid + NW, *ibufs[1])

    def step(g, carry):
        for k in range(2):
            b = wid + (2 * g + k) * NW
            finish(b, *ibufs[k])

            @pl.when(b + 2 * NW < N_BLOCKS)
            def _():
                start(b + 2 * NW, *ibufs[k])

        return carry

    lax.fori_loop(0, N_BLOCKS // NW // 2, step, 0)

    @pl.when(wid < N_BLOCKS - (N_BLOCKS // NW) * NW)
    def _():
        finish(wid + (N_BLOCKS // NW) * NW, *ibufs[0])

    plsc.subcore_barrier()

    out_base = cid * R_PAD + base
    for k in range(RES_PER_SUB // BLK_ATOMS):
        pltpu.sync_copy(ccnt.at[pl.ds(base + k * BLK_ATOMS, BLK_ATOMS)], ones_v)
        pltpu.sync_copy(ones_v, cnts_out.at[pl.ds(out_base + k * BLK_ATOMS, BLK_ATOMS)])


@jax.jit
def _sc_segment_sums(atoms, map1d, zrows, ones_in):
    mesh = plsc.VectorSubcoreMesh(core_axis_name="c", subcore_axis_name="s",
                                  num_cores=NC, num_subcores=NS)
    sums = pl.kernel(
        _sc_sums_body,
        out_type=[
            jax.ShapeDtypeStruct((NC * R_PAD, ATOM_DIM), jnp.float32),
        ],
        mesh=mesh,
        scratch_types=[
            pltpu.VMEM((BLK_ATOMS, ATOM_DIM), jnp.float32),
            pltpu.VMEM((BLK_ATOMS, ATOM_DIM), jnp.float32),
            pltpu.VMEM((BLK_ATOMS,), jnp.int32),
            pltpu.VMEM((BLK_ATOMS,), jnp.int32),
            pltpu.SemaphoreType.DMA,
            pltpu.SemaphoreType.DMA,
            pltpu.SemaphoreType.DMA,
            pltpu.SemaphoreType.DMA,
            pltpu.VMEM_SHARED((R_PAD, ATOM_DIM), jnp.float32),
        ],
    )(atoms, map1d, zrows)[0]
    cnts = pl.kernel(
        _sc_counts_body,
        out_type=[
            jax.ShapeDtypeStruct((NC * R_PAD, ATOM_DIM), jnp.float32),
        ],
        mesh=mesh,
        scratch_types=[
            pltpu.VMEM((BLK_ATOMS, ATOM_DIM), jnp.float32),
            pltpu.VMEM((BLK_ATOMS,), jnp.int32),
            pltpu.VMEM((BLK_ATOMS,), jnp.int32),
            pltpu.SemaphoreType.DMA,
            pltpu.SemaphoreType.DMA,
            pltpu.VMEM_SHARED((R_PAD, ATOM_DIM), jnp.float32),
        ],
    )(map1d, zrows, ones_in)[0]
    return sums, cnts


def _mlp_body(sums_ref, cnts_ref, w1t_ref, b1_ref, w2t_ref, b2_ref, out_ref):
    s = sums_ref[0] + sums_ref[1]
    c = cnts_ref[0] + cnts_ref[1]
    r = s / c
    h = jnp.dot(r, w1t_ref[...], preferred_element_type=jnp.float32) + b1_ref[...]
    h = jnp.where(h >= 0.0, h, 0.01 * h)
    out_ref[...] = (jnp.dot(h, w2t_ref[...], preferred_element_type=jnp.float32)
                    + b2_ref[...])


MLP_BLK = 1000


@jax.jit
def _tc_mlp(sums, cnts, w1t, b1, w2t, b2):
    grid = (N_RES // MLP_BLK,)
    return pl.pallas_call(
        _mlp_body,
        grid=grid,
        in_specs=[
            pl.BlockSpec((NC, MLP_BLK, ATOM_DIM), lambda i: (0, i, 0)),
            pl.BlockSpec((NC, MLP_BLK, 1), lambda i: (0, i, 0)),
            pl.BlockSpec((ATOM_DIM, HID), lambda i: (0, 0)),
            pl.BlockSpec((1, HID), lambda i: (0, 0)),
            pl.BlockSpec((HID, OUT_DIM), lambda i: (0, 0)),
            pl.BlockSpec((1, OUT_DIM), lambda i: (0, 0)),
        ],
        out_specs=pl.BlockSpec((MLP_BLK, OUT_DIM), lambda i: (i, 0)),
        out_shape=jax.ShapeDtypeStruct((N_RES, OUT_DIM), jnp.float32),
    )(sums, cnts, w1t, b1, w2t, b2)


def kernel(atom_features, residue_mapping, W1, b1, W2, b2):
    map1d = residue_mapping.astype(jnp.int32)
    zrows = jnp.zeros((BLK_ATOMS, ATOM_DIM), jnp.float32)
    ones_in = jnp.ones((BLK_ATOMS, ATOM_DIM), jnp.float32)
    sums, cnts = _sc_segment_sums(atom_features, map1d, zrows, ones_in)
    sums = sums.reshape(NC, R_PAD, ATOM_DIM)[:, :N_RES, :]
    cnts = cnts.reshape(NC, R_PAD, ATOM_DIM)[:, :N_RES, :1]
    out = _tc_mlp(sums, cnts,
                  W1.T, b1.reshape(1, HID), W2.T, b2.reshape(1, OUT_DIM))
    return out


# speedup vs baseline: 7.0072x; 1.2157x over previous
import jax
import jax.numpy as jnp
from jax import lax
from jax.experimental import pallas as pl
from jax.experimental.pallas import tpu as pltpu
from jax.experimental.pallas import tpu_sc as plsc

ATOM_DIM = 128
HID = 256
OUT_DIM = 20
N_ATOMS = 320000
N_RES = 10000
BLK_ATOMS = 128
N_BLOCKS = N_ATOMS // BLK_ATOMS
NC = 2
NS = 16
NW = NC * NS
R_PAD = 10240
CNT_W = 64
RES_PER_SUB = R_PAD // NS


def _sc_sums_body(atoms, map1d, zrows, sums_out,
                  d0, d1, i0, i1, sd0, sd1, si0, si1, acc):
    cid = lax.axis_index("c")
    sid = lax.axis_index("s")
    wid = cid * NS + sid
    bufs = ((d0, i0, sd0, si0), (d1, i1, sd1, si1))

    base = sid * RES_PER_SUB
    pltpu.sync_copy(zrows, d0)
    for k in range(RES_PER_SUB // BLK_ATOMS):
        pltpu.sync_copy(d0, acc.at[pl.ds(base + k * BLK_ATOMS, BLK_ATOMS)])
    plsc.subcore_barrier()

    def start(b, buf):
        dv, iv, sd, si = buf
        pltpu.async_copy(atoms.at[pl.ds(b * BLK_ATOMS, BLK_ATOMS)], dv, sd)
        pltpu.async_copy(map1d.at[pl.ds(b * BLK_ATOMS, BLK_ATOMS)], iv, si)

    def finish(b, buf):
        dv, iv, sd, si = buf
        pltpu.make_async_copy(atoms.at[pl.ds(b * BLK_ATOMS, BLK_ATOMS)], dv, sd).wait()
        pltpu.make_async_copy(map1d.at[pl.ds(b * BLK_ATOMS, BLK_ATOMS)], iv, si).wait()
        pltpu.sync_copy(dv, acc.at[iv], add=True)

    start(wid, bufs[0])
    start(wid + NW, bufs[1])

    def step(g, carry):
        for k in range(2):
            b = wid + (2 * g + k) * NW
            finish(b, bufs[k])

            @pl.when(b + 2 * NW < N_BLOCKS)
            def _():
                start(b + 2 * NW, bufs[k])

        return carry

    lax.fori_loop(0, N_BLOCKS // NW // 2, step, 0)

    @pl.when(wid < N_BLOCKS - (N_BLOCKS // NW) * NW)
    def _():
        finish(wid + (N_BLOCKS // NW) * NW, bufs[0])

    plsc.subcore_barrier()

    out_base = cid * R_PAD + base
    for k in range(RES_PER_SUB // BLK_ATOMS):
        pltpu.sync_copy(acc.at[pl.ds(base + k * BLK_ATOMS, BLK_ATOMS)], d0)
        pltpu.sync_copy(d0, sums_out.at[pl.ds(out_base + k * BLK_ATOMS, BLK_ATOMS)])


def _sc_counts_body(map1d, zcnt, ones_in, cnts_out, ones_v, i0, i1, si0, si1, ccnt):
    cid = lax.axis_index("c")
    sid = lax.axis_index("s")
    wid = cid * NS + sid

    base = sid * RES_PER_SUB
    pltpu.sync_copy(zcnt, ones_v)
    for k in range(RES_PER_SUB // BLK_ATOMS):
        pltpu.sync_copy(ones_v, ccnt.at[pl.ds(base + k * BLK_ATOMS, BLK_ATOMS)])
    pltpu.sync_copy(ones_in, ones_v)
    plsc.subcore_barrier()

    def start(b, iv, si):
        pltpu.async_copy(map1d.at[pl.ds(b * BLK_ATOMS, BLK_ATOMS)], iv, si)

    def finish(b, iv, si):
        pltpu.make_async_copy(map1d.at[pl.ds(b * BLK_ATOMS, BLK_ATOMS)], iv, si).wait()
        pltpu.sync_copy(ones_v, ccnt.at[iv], add=True)

    ibufs = ((i0, si0), (i1, si1))
    start(wid, *ibufs[0])
    start(wid + NW, *ibufs[1])

    def step(g, carry):
        for k in range(2):
            b = wid + (2 * g + k) * NW
            finish(b, *ibufs[k])

            @pl.when(b + 2 * NW < N_BLOCKS)
            def _():
                start(b + 2 * NW, *ibufs[k])

        return carry

    lax.fori_loop(0, N_BLOCKS // NW // 2, step, 0)

    @pl.when(wid < N_BLOCKS - (N_BLOCKS // NW) * NW)
    def _():
        finish(wid + (N_BLOCKS // NW) * NW, *ibufs[0])

    plsc.subcore_barrier()

    out_base = cid * R_PAD + base
    for k in range(RES_PER_SUB // BLK_ATOMS):
        pltpu.sync_copy(ccnt.at[pl.ds(base + k * BLK_ATOMS, BLK_ATOMS)], ones_v)
        pltpu.sync_copy(ones_v, cnts_out.at[pl.ds(out_base + k * BLK_ATOMS, BLK_ATOMS)])


@jax.jit
def _sc_segment_sums(atoms, map1d, zrows, zcnt, ones_cnt):
    mesh = plsc.VectorSubcoreMesh(core_axis_name="c", subcore_axis_name="s",
                                  num_cores=NC, num_subcores=NS)
    sums = pl.kernel(
        _sc_sums_body,
        out_type=[
            jax.ShapeDtypeStruct((NC * R_PAD, ATOM_DIM), jnp.float32),
        ],
        mesh=mesh,
        scratch_types=[
            pltpu.VMEM((BLK_ATOMS, ATOM_DIM), jnp.float32),
            pltpu.VMEM((BLK_ATOMS, ATOM_DIM), jnp.float32),
            pltpu.VMEM((BLK_ATOMS,), jnp.int32),
            pltpu.VMEM((BLK_ATOMS,), jnp.int32),
            pltpu.SemaphoreType.DMA,
            pltpu.SemaphoreType.DMA,
            pltpu.SemaphoreType.DMA,
            pltpu.SemaphoreType.DMA,
            pltpu.VMEM_SHARED((R_PAD, ATOM_DIM), jnp.float32),
        ],
    )(atoms, map1d, zrows)[0]
    cnts = pl.kernel(
        _sc_counts_body,
        out_type=[
            jax.ShapeDtypeStruct((NC * R_PAD, CNT_W), jnp.float32),
        ],
        mesh=mesh,
        scratch_types=[
            pltpu.VMEM((BLK_ATOMS, CNT_W), jnp.float32),
            pltpu.VMEM((BLK_ATOMS,), jnp.int32),
            pltpu.VMEM((BLK_ATOMS,), jnp.int32),
            pltpu.SemaphoreType.DMA,
            pltpu.SemaphoreType.DMA,
            pltpu.VMEM_SHARED((R_PAD, CNT_W), jnp.float32),
        ],
    )(map1d, zcnt, ones_cnt)[0]
    return sums, cnts


def _mlp_body(sums_ref, cnts_ref, w1t_ref, b1_ref, w2t_ref, b2_ref, out_ref):
    s = sums_ref[0] + sums_ref[1]
    c = (cnts_ref[0] + cnts_ref[1])[:, :1]
    r = s / c
    h = jnp.dot(r, w1t_ref[...], preferred_element_type=jnp.float32) + b1_ref[...]
    h = jnp.where(h >= 0.0, h, 0.01 * h)
    out_ref[...] = (jnp.dot(h, w2t_ref[...], preferred_element_type=jnp.float32)
                    + b2_ref[...])


MLP_BLK = 1000


@jax.jit
def _tc_mlp(sums, cnts, w1t, b1, w2t, b2):
    grid = (N_RES // MLP_BLK,)
    return pl.pallas_call(
        _mlp_body,
        grid=grid,
        in_specs=[
            pl.BlockSpec((NC, MLP_BLK, ATOM_DIM), lambda i: (0, i, 0)),
            pl.BlockSpec((NC, MLP_BLK, CNT_W), lambda i: (0, i, 0)),
            pl.BlockSpec((ATOM_DIM, HID), lambda i: (0, 0)),
            pl.BlockSpec((1, HID), lambda i: (0, 0)),
            pl.BlockSpec((HID, OUT_DIM), lambda i: (0, 0)),
            pl.BlockSpec((1, OUT_DIM), lambda i: (0, 0)),
        ],
        out_specs=pl.BlockSpec((MLP_BLK, OUT_DIM), lambda i: (i, 0)),
        out_shape=jax.ShapeDtypeStruct((N_RES, OUT_DIM), jnp.float32),
    )(sums, cnts, w1t, b1, w2t, b2)


def kernel(atom_features, residue_mapping, W1, b1, W2, b2):
    map1d = residue_mapping.astype(jnp.int32)
    zrows = jnp.zeros((BLK_ATOMS, ATOM_DIM), jnp.float32)
    zcnt = jnp.zeros((BLK_ATOMS, CNT_W), jnp.float32)
    ones_cnt = jnp.ones((BLK_ATOMS, CNT_W), jnp.float32)
    sums, cnts = _sc_segment_sums(atom_features, map1d, zrows, zcnt, ones_cnt)
    sums = sums.reshape(NC, R_PAD, ATOM_DIM)
    cnts = cnts.reshape(NC, R_PAD, CNT_W)
    out = _tc_mlp(sums, cnts,
                  W1.T, b1.reshape(1, HID), W2.T, b2.reshape(1, OUT_DIM))
    return out


# speedup vs baseline: 7.3446x; 1.0481x over previous
import jax
import jax.numpy as jnp
from jax import lax
from jax.experimental import pallas as pl
from jax.experimental.pallas import tpu as pltpu
from jax.experimental.pallas import tpu_sc as plsc

ATOM_DIM = 128
HID = 256
OUT_DIM = 20
N_ATOMS = 320000
N_RES = 10000
BLK_ATOMS = 128
N_BLOCKS = N_ATOMS // BLK_ATOMS
NC = 2
NS = 16
NW = NC * NS
R_PAD = 10240
CNT_W = 32
RES_PER_SUB = R_PAD // NS


def _sc_sums_body(atoms, map1d, zrows, sums_out,
                  d0, d1, i0, i1, sd0, sd1, si0, si1, acc):
    cid = lax.axis_index("c")
    sid = lax.axis_index("s")
    wid = cid * NS + sid
    bufs = ((d0, i0, sd0, si0), (d1, i1, sd1, si1))

    base = sid * RES_PER_SUB
    pltpu.sync_copy(zrows, d0)
    for k in range(RES_PER_SUB // BLK_ATOMS):
        pltpu.sync_copy(d0, acc.at[pl.ds(base + k * BLK_ATOMS, BLK_ATOMS)])
    plsc.subcore_barrier()

    def start(b, buf):
        dv, iv, sd, si = buf
        pltpu.async_copy(atoms.at[pl.ds(b * BLK_ATOMS, BLK_ATOMS)], dv, sd)
        pltpu.async_copy(map1d.at[pl.ds(b * BLK_ATOMS, BLK_ATOMS)], iv, si)

    def finish(b, buf):
        dv, iv, sd, si = buf
        pltpu.make_async_copy(atoms.at[pl.ds(b * BLK_ATOMS, BLK_ATOMS)], dv, sd).wait()
        pltpu.make_async_copy(map1d.at[pl.ds(b * BLK_ATOMS, BLK_ATOMS)], iv, si).wait()
        pltpu.sync_copy(dv, acc.at[iv], add=True)

    start(wid, bufs[0])
    start(wid + NW, bufs[1])

    def step(g, carry):
        for k in range(2):
            b = wid + (2 * g + k) * NW
            finish(b, bufs[k])

            @pl.when(b + 2 * NW < N_BLOCKS)
            def _():
                start(b + 2 * NW, bufs[k])

        return carry

    lax.fori_loop(0, N_BLOCKS // NW // 2, step, 0)

    @pl.when(wid < N_BLOCKS - (N_BLOCKS // NW) * NW)
    def _():
        finish(wid + (N_BLOCKS // NW) * NW, bufs[0])

    plsc.subcore_barrier()

    out_base = cid * R_PAD + base
    for k in range(RES_PER_SUB // BLK_ATOMS):
        pltpu.sync_copy(acc.at[pl.ds(base + k * BLK_ATOMS, BLK_ATOMS)], d0)
        pltpu.sync_copy(d0, sums_out.at[pl.ds(out_base + k * BLK_ATOMS, BLK_ATOMS)])


def _sc_counts_body(map1d, zcnt, ones_in, cnts_out, ones_v, i0, i1, si0, si1, ccnt):
    cid = lax.axis_index("c")
    sid = lax.axis_index("s")
    wid = cid * NS + sid

    base = sid * RES_PER_SUB
    pltpu.sync_copy(zcnt, ones_v)
    for k in range(RES_PER_SUB // BLK_ATOMS):
        pltpu.sync_copy(ones_v, ccnt.at[pl.ds(base + k * BLK_ATOMS, BLK_ATOMS)])
    pltpu.sync_copy(ones_in, ones_v)
    plsc.subcore_barrier()

    def start(b, iv, si):
        pltpu.async_copy(map1d.at[pl.ds(b * BLK_ATOMS, BLK_ATOMS)], iv, si)

    def finish(b, iv, si):
        pltpu.make_async_copy(map1d.at[pl.ds(b * BLK_ATOMS, BLK_ATOMS)], iv, si).wait()
        pltpu.sync_copy(ones_v, ccnt.at[iv], add=True)

    ibufs = ((i0, si0), (i1, si1))
    start(wid, *ibufs[0])
    start(wid + NW, *ibufs[1])

    def step(g, carry):
        for k in range(2):
            b = wid + (2 * g + k) * NW
            finish(b, *ibufs[k])

            @pl.when(b + 2 * NW < N_BLOCKS)
            def _():
                start(b + 2 * NW, *ibufs[k])

        return carry

    lax.fori_loop(0, N_BLOCKS // NW // 2, step, 0)

    @pl.when(wid < N_BLOCKS - (N_BLOCKS // NW) * NW)
    def _():
        finish(wid + (N_BLOCKS // NW) * NW, *ibufs[0])

    plsc.subcore_barrier()

    out_base = cid * R_PAD + base
    for k in range(RES_PER_SUB // BLK_ATOMS):
        pltpu.sync_copy(ccnt.at[pl.ds(base + k * BLK_ATOMS, BLK_ATOMS)], ones_v)
        pltpu.sync_copy(ones_v, cnts_out.at[pl.ds(out_base + k * BLK_ATOMS, BLK_ATOMS)])


@jax.jit
def _sc_segment_sums(atoms, map1d, zrows, zcnt, ones_cnt):
    mesh = plsc.VectorSubcoreMesh(core_axis_name="c", subcore_axis_name="s",
                                  num_cores=NC, num_subcores=NS)
    sums = pl.kernel(
        _sc_sums_body,
        out_type=[
            jax.ShapeDtypeStruct((NC * R_PAD, ATOM_DIM), jnp.float32),
        ],
        mesh=mesh,
        scratch_types=[
            pltpu.VMEM((BLK_ATOMS, ATOM_DIM), jnp.float32),
            pltpu.VMEM((BLK_ATOMS, ATOM_DIM), jnp.float32),
            pltpu.VMEM((BLK_ATOMS,), jnp.int32),
            pltpu.VMEM((BLK_ATOMS,), jnp.int32),
            pltpu.SemaphoreType.DMA,
            pltpu.SemaphoreType.DMA,
            pltpu.SemaphoreType.DMA,
            pltpu.SemaphoreType.DMA,
            pltpu.VMEM_SHARED((R_PAD, ATOM_DIM), jnp.float32),
        ],
    )(atoms, map1d, zrows)[0]
    cnts = pl.kernel(
        _sc_counts_body,
        out_type=[
            jax.ShapeDtypeStruct((NC * R_PAD, CNT_W), jnp.float32),
        ],
        mesh=mesh,
        scratch_types=[
            pltpu.VMEM((BLK_ATOMS, CNT_W), jnp.float32),
            pltpu.VMEM((BLK_ATOMS,), jnp.int32),
            pltpu.VMEM((BLK_ATOMS,), jnp.int32),
            pltpu.SemaphoreType.DMA,
            pltpu.SemaphoreType.DMA,
            pltpu.VMEM_SHARED((R_PAD, CNT_W), jnp.float32),
        ],
    )(map1d, zcnt, ones_cnt)[0]
    return sums, cnts


def _mlp_body(sums_ref, cnts_ref, w1t_ref, b1_ref, w2t_ref, b2_ref, out_ref):
    s = sums_ref[0] + sums_ref[1]
    c = (cnts_ref[0] + cnts_ref[1])[:, :1]
    r = s / c
    h = jnp.dot(r, w1t_ref[...], preferred_element_type=jnp.float32) + b1_ref[...]
    h = jnp.where(h >= 0.0, h, 0.01 * h)
    out_ref[...] = (jnp.dot(h, w2t_ref[...], preferred_element_type=jnp.float32)
                    + b2_ref[...])


MLP_BLK = 2000


@jax.jit
def _tc_mlp(sums, cnts, w1t, b1, w2t, b2):
    grid = (N_RES // MLP_BLK,)
    return pl.pallas_call(
        _mlp_body,
        grid=grid,
        in_specs=[
            pl.BlockSpec((NC, MLP_BLK, ATOM_DIM), lambda i: (0, i, 0)),
            pl.BlockSpec((NC, MLP_BLK, CNT_W), lambda i: (0, i, 0)),
            pl.BlockSpec((ATOM_DIM, HID), lambda i: (0, 0)),
            pl.BlockSpec((1, HID), lambda i: (0, 0)),
            pl.BlockSpec((HID, OUT_DIM), lambda i: (0, 0)),
            pl.BlockSpec((1, OUT_DIM), lambda i: (0, 0)),
        ],
        out_specs=pl.BlockSpec((MLP_BLK, OUT_DIM), lambda i: (i, 0)),
        out_shape=jax.ShapeDtypeStruct((N_RES, OUT_DIM), jnp.float32),
    )(sums, cnts, w1t, b1, w2t, b2)


def kernel(atom_features, residue_mapping, W1, b1, W2, b2):
    map1d = residue_mapping.astype(jnp.int32)
    zrows = jnp.zeros((BLK_ATOMS, ATOM_DIM), jnp.float32)
    zcnt = jnp.zeros((BLK_ATOMS, CNT_W), jnp.float32)
    ones_cnt = jnp.ones((BLK_ATOMS, CNT_W), jnp.float32)
    sums, cnts = _sc_segment_sums(atom_features, map1d, zrows, zcnt, ones_cnt)
    sums = sums.reshape(NC, R_PAD, ATOM_DIM)
    cnts = cnts.reshape(NC, R_PAD, CNT_W)
    out = _tc_mlp(sums, cnts,
                  W1.T, b1.reshape(1, HID), W2.T, b2.reshape(1, OUT_DIM))
    return out


# speedup vs baseline: 7.8666x; 1.0711x over previous
import jax
import jax.numpy as jnp
from jax import lax
from jax.experimental import pallas as pl
from jax.experimental.pallas import tpu as pltpu
from jax.experimental.pallas import tpu_sc as plsc

ATOM_DIM = 128
HID = 256
OUT_DIM = 20
N_ATOMS = 320000
N_RES = 10000
BLK_ATOMS = 128
N_BLOCKS = N_ATOMS // BLK_ATOMS
NC = 2
NS = 16
NW = NC * NS
R_PAD = 10112
CNT_W = 32
RES_PER_SUB = R_PAD // NS
CHUNKS = ((0, 128), (128, 128), (256, 128), (384, 128), (512, 120))


def _sc_sums_body(atoms, map1d, zrows, sums_out,
                  d0, d1, d2, i0, i1, i2, sl0, sl1, sl2, ss0, ss1, ss2, acc):
    cid = lax.axis_index("c")
    sid = lax.axis_index("s")
    wid = cid * NS + sid
    bufs = ((d0, i0, sl0, ss0), (d1, i1, sl1, ss1), (d2, i2, sl2, ss2))

    base = sid * RES_PER_SUB
    pltpu.sync_copy(zrows, d0)
    for off, sz in CHUNKS:
        pltpu.sync_copy(d0.at[pl.ds(0, sz)], acc.at[pl.ds(base + off, sz)])
    plsc.subcore_barrier()

    def start_loads(b, buf):
        dv, iv, sl, _ = buf
        pltpu.async_copy(atoms.at[pl.ds(b * BLK_ATOMS, BLK_ATOMS)], dv, sl)
        pltpu.async_copy(map1d.at[pl.ds(b * BLK_ATOMS, BLK_ATOMS)], iv, sl)

    def wait_loads(b, buf):
        dv, iv, sl, _ = buf
        pltpu.make_async_copy(atoms.at[pl.ds(b * BLK_ATOMS, BLK_ATOMS)], dv, sl).wait()
        pltpu.make_async_copy(map1d.at[pl.ds(b * BLK_ATOMS, BLK_ATOMS)], iv, sl).wait()

    for k in range(3):
        start_loads(wid + k * NW, bufs[k])

    def step(g, carry):
        for k in range(3):
            b = wid + (3 * g + k) * NW
            dv, iv, _, ss = bufs[k]
            wait_loads(b, bufs[k])
            pltpu.async_copy(dv, acc.at[iv], ss, add=True)
            nxt = b + 3 * NW

            @pl.when(nxt < N_BLOCKS)
            def _():
                pltpu.make_async_copy(dv, acc.at[iv], ss).wait()
                start_loads(nxt, bufs[k])

        return carry

    lax.fori_loop(0, N_BLOCKS // NW // 3, step, 0)

    for k in range(3):
        b_last = wid + (N_BLOCKS // NW - 3 + k) * NW
        dv, iv, _, ss = bufs[k]

        @pl.when(b_last + 3 * NW >= N_BLOCKS)
        def _():
            pltpu.make_async_copy(dv, acc.at[iv], ss).wait()

    @pl.when(wid < N_BLOCKS - (N_BLOCKS // NW) * NW)
    def _():
        wait_loads(wid + (N_BLOCKS // NW) * NW, bufs[0])
        pltpu.sync_copy(d0, acc.at[i0], add=True)

    plsc.subcore_barrier()

    out_base = cid * R_PAD + base
    for off, sz in CHUNKS:
        pltpu.sync_copy(acc.at[pl.ds(base + off, sz)], d0.at[pl.ds(0, sz)])
        pltpu.sync_copy(d0.at[pl.ds(0, sz)], sums_out.at[pl.ds(out_base + off, sz)])


def _sc_counts_body(map1d, zcnt, ones_in, cnts_out,
                    ones_v, i0, i1, i2, sl0, sl1, sl2, ss0, ss1, ss2, ccnt):
    cid = lax.axis_index("c")
    sid = lax.axis_index("s")
    wid = cid * NS + sid
    bufs = ((i0, sl0, ss0), (i1, sl1, ss1), (i2, sl2, ss2))

    base = sid * RES_PER_SUB
    pltpu.sync_copy(zcnt, ones_v)
    for off, sz in CHUNKS:
        pltpu.sync_copy(ones_v.at[pl.ds(0, sz)], ccnt.at[pl.ds(base + off, sz)])
    pltpu.sync_copy(ones_in, ones_v)
    plsc.subcore_barrier()

    def start_load(b, buf):
        iv, sl, _ = buf
        pltpu.async_copy(map1d.at[pl.ds(b * BLK_ATOMS, BLK_ATOMS)], iv, sl)

    def wait_load(b, buf):
        iv, sl, _ = buf
        pltpu.make_async_copy(map1d.at[pl.ds(b * BLK_ATOMS, BLK_ATOMS)], iv, sl).wait()

    for k in range(3):
        start_load(wid + k * NW, bufs[k])

    def step(g, carry):
        for k in range(3):
            b = wid + (3 * g + k) * NW
            iv, _, ss = bufs[k]
            wait_load(b, bufs[k])
            pltpu.async_copy(ones_v, ccnt.at[iv], ss, add=True)
            nxt = b + 3 * NW

            @pl.when(nxt < N_BLOCKS)
            def _():
                pltpu.make_async_copy(ones_v, ccnt.at[iv], ss).wait()
                start_load(nxt, bufs[k])

        return carry

    lax.fori_loop(0, N_BLOCKS // NW // 3, step, 0)

    for k in range(3):
        b_last = wid + (N_BLOCKS // NW - 3 + k) * NW
        iv, _, ss = bufs[k]

        @pl.when(b_last + 3 * NW >= N_BLOCKS)
        def _():
            pltpu.make_async_copy(ones_v, ccnt.at[iv], ss).wait()

    @pl.when(wid < N_BLOCKS - (N_BLOCKS // NW) * NW)
    def _():
        wait_load(wid + (N_BLOCKS // NW) * NW, bufs[0])
        pltpu.sync_copy(ones_v, ccnt.at[i0], add=True)

    plsc.subcore_barrier()

    out_base = cid * R_PAD + base
    for off, sz in CHUNKS:
        pltpu.sync_copy(ccnt.at[pl.ds(base + off, sz)], ones_v.at[pl.ds(0, sz)])
        pltpu.sync_copy(ones_v.at[pl.ds(0, sz)], cnts_out.at[pl.ds(out_base + off, sz)])


@jax.jit
def _sc_segment_sums(atoms, map1d, zrows, zcnt, ones_cnt):
    mesh = plsc.VectorSubcoreMesh(core_axis_name="c", subcore_axis_name="s",
                                  num_cores=NC, num_subcores=NS)
    sums = pl.kernel(
        _sc_sums_body,
        out_type=[
            jax.ShapeDtypeStruct((NC * R_PAD, ATOM_DIM), jnp.float32),
        ],
        mesh=mesh,
        scratch_types=[
            pltpu.VMEM((BLK_ATOMS, ATOM_DIM), jnp.float32),
            pltpu.VMEM((BLK_ATOMS, ATOM_DIM), jnp.float32),
            pltpu.VMEM((BLK_ATOMS, ATOM_DIM), jnp.float32),
            pltpu.VMEM((BLK_ATOMS,), jnp.int32),
            pltpu.VMEM((BLK_ATOMS,), jnp.int32),
            pltpu.VMEM((BLK_ATOMS,), jnp.int32),
            pltpu.SemaphoreType.DMA,
            pltpu.SemaphoreType.DMA,
            pltpu.SemaphoreType.DMA,
            pltpu.SemaphoreType.DMA,
            pltpu.SemaphoreType.DMA,
            pltpu.SemaphoreType.DMA,
            pltpu.VMEM_SHARED((R_PAD, ATOM_DIM), jnp.float32),
        ],
    )(atoms, map1d, zrows)[0]
    cnts = pl.kernel(
        _sc_counts_body,
        out_type=[
            jax.ShapeDtypeStruct((NC * R_PAD, CNT_W), jnp.float32),
        ],
        mesh=mesh,
        scratch_types=[
            pltpu.VMEM((BLK_ATOMS, CNT_W), jnp.float32),
            pltpu.VMEM((BLK_ATOMS,), jnp.int32),
            pltpu.VMEM((BLK_ATOMS,), jnp.int32),
            pltpu.VMEM((BLK_ATOMS,), jnp.int32),
            pltpu.SemaphoreType.DMA,
            pltpu.SemaphoreType.DMA,
            pltpu.SemaphoreType.DMA,
            pltpu.SemaphoreType.DMA,
            pltpu.SemaphoreType.DMA,
            pltpu.SemaphoreType.DMA,
            pltpu.VMEM_SHARED((R_PAD, CNT_W), jnp.float32),
        ],
    )(map1d, zcnt, ones_cnt)[0]
    return sums, cnts


def _mlp_body(sums_ref, cnts_ref, w1t_ref, b1_ref, w2t_ref, b2_ref, out_ref):
    s = sums_ref[0] + sums_ref[1]
    c = (cnts_ref[0] + cnts_ref[1])[:, :1]
    r = s / c
    h = jnp.dot(r, w1t_ref[...], preferred_element_type=jnp.float32) + b1_ref[...]
    h = jnp.where(h >= 0.0, h, 0.01 * h)
    out_ref[...] = (jnp.dot(h, w2t_ref[...], preferred_element_type=jnp.float32)
                    + b2_ref[...])


MLP_BLK = 2000


@jax.jit
def _tc_mlp(sums, cnts, w1t, b1, w2t, b2):
    grid = (N_RES // MLP_BLK,)
    return pl.pallas_call(
        _mlp_body,
        grid=grid,
        in_specs=[
            pl.BlockSpec((NC, MLP_BLK, ATOM_DIM), lambda i: (0, i, 0)),
            pl.BlockSpec((NC, MLP_BLK, CNT_W), lambda i: (0, i, 0)),
            pl.BlockSpec((ATOM_DIM, HID), lambda i: (0, 0)),
            pl.BlockSpec((1, HID), lambda i: (0, 0)),
            pl.BlockSpec((HID, OUT_DIM), lambda i: (0, 0)),
            pl.BlockSpec((1, OUT_DIM), lambda i: (0, 0)),
        ],
        out_specs=pl.BlockSpec((MLP_BLK, OUT_DIM), lambda i: (i, 0)),
        out_shape=jax.ShapeDtypeStruct((N_RES, OUT_DIM), jnp.float32),
    )(sums, cnts, w1t, b1, w2t, b2)


def kernel(atom_features, residue_mapping, W1, b1, W2, b2):
    map1d = residue_mapping.astype(jnp.int32)
    zrows = jnp.zeros((BLK_ATOMS, ATOM_DIM), jnp.float32)
    zcnt = jnp.zeros((BLK_ATOMS, CNT_W), jnp.float32)
    ones_cnt = jnp.ones((BLK_ATOMS, CNT_W), jnp.float32)
    sums, cnts = _sc_segment_sums(atom_features, map1d, zrows, zcnt, ones_cnt)
    sums = sums.reshape(NC, R_PAD, ATOM_DIM)
    cnts = cnts.reshape(NC, R_PAD, CNT_W)
    out = _tc_mlp(sums, cnts,
                  W1.T, b1.reshape(1, HID), W2.T, b2.reshape(1, OUT_DIM))
    return out
